# baseline (speedup 1.0000x reference)
import jax
import jax.numpy as jnp
from jax.experimental import pallas as pl
from jax.experimental.pallas import tpu as pltpu

C = 384
C3 = 3 * C
NE = 20
B = 8
H = 24
W = 24
ROWS = H * W
S = 640
G = 32
PBUF = B * S
ADT = jnp.bfloat16


def _row_masks():
    r = jax.lax.broadcasted_iota(jnp.int32, (ROWS, 1), 0)
    mask_a = jnp.where(r % W == W - 1, 0.0, 1.0)
    mask_b = jnp.where(r % W == 0, 0.0, 1.0)
    return mask_a, mask_b


def _zero_pads(ref):
    z = jnp.zeros((40, C3), ADT)
    for b in range(B):
        ref[pl.ds(b * S, 40), :] = z
        ref[pl.ds(b * S + S - 40, 40), :] = z


def _store_tri(dst, base, val, mask_a, mask_b):
    dst[pl.ds(base + G, ROWS), 0:C] = val.astype(ADT)
    dst[pl.ds(base + G + 1, ROWS), C:2 * C] = (val * mask_a).astype(ADT)
    dst[pl.ds(base + G - 1, ROWS), 2 * C:C3] = (val * mask_b).astype(ADT)


def _conv(src, dst, wk, kj, bias, mask_a, mask_b, resid_src=None,
          out_ref=None):
    for b in range(B):
        base = b * S
        lm = src[pl.ds(base + G - W, ROWS), :]
        lc = src[pl.ds(base + G, ROWS), :]
        lp = src[pl.ds(base + G + W, ROWS), :]
        acc = jnp.dot(lm, wk[kj, 0], preferred_element_type=jnp.float32)
        acc = acc + jnp.dot(lc, wk[kj, 1],
                            preferred_element_type=jnp.float32)
        acc = acc + jnp.dot(lp, wk[kj, 2],
                            preferred_element_type=jnp.float32)
        val = acc + bias
        if resid_src is not None:
            val = val + resid_src[pl.ds(base + G, ROWS), 0:C].astype(
                jnp.float32)
        val = jnp.maximum(val, 0.0)
        if out_ref is None:
            _store_tri(dst, base, val, mask_a, mask_b)
        else:
            out_ref[b] = val.reshape(H, W, C)


def _body0(xt, wmap_t, projw_t, pb, gum, emb, wk, rb, h_out, r_scr):
    mask_a, mask_b = _row_masks()
    xv = xt[...].reshape(B, ROWS, C)
    pooled = jnp.mean(xv, axis=1)
    m = jax.nn.sigmoid(jnp.dot(pooled, wmap_t[...],
                               preferred_element_type=jnp.float32))
    logits = jnp.dot(m, projw_t[...],
                     preferred_element_type=jnp.float32) + pb[...]
    y = logits + gum[...]
    col = jax.lax.broadcasted_iota(jnp.int32, (B, NE), 1)
    ymax = jnp.max(y, axis=1, keepdims=True)
    amin = jnp.min(jnp.where(y == ymax, col, NE), axis=1, keepdims=True)
    oh = (col == amin).astype(jnp.float32)
    zq = jnp.dot(oh, emb[...], preferred_element_type=jnp.float32)
    v = xv + zq[:, None, :]
    _zero_pads(h_out)
    _zero_pads(r_scr)
    for b in range(B):
        _store_tri(h_out, b * S, v[b], mask_a, mask_b)
    _conv(h_out, r_scr, wk, 0, rb[0][None, :], mask_a, mask_b)
    _conv(r_scr, h_out, wk, 1, rb[1][None, :], mask_a, mask_b,
          resid_src=h_out)


def _body1(h_in, wk, rb, h_out, r_scr):
    mask_a, mask_b = _row_masks()
    _zero_pads(h_out)
    _zero_pads(r_scr)
    _conv(h_in, r_scr, wk, 0, rb[0][None, :], mask_a, mask_b)
    _conv(r_scr, h_out, wk, 1, rb[1][None, :], mask_a, mask_b,
          resid_src=h_in)


def _body2(h_in, wk, rb, out, r_scr):
    mask_a, mask_b = _row_masks()
    _zero_pads(r_scr)
    _conv(h_in, r_scr, wk, 0, rb[0][None, :], mask_a, mask_b)
    _conv(r_scr, None, wk, 1, rb[1][None, :], mask_a, mask_b,
          resid_src=h_in, out_ref=out)


def _call0(interpret=False):
    return pl.pallas_call(
        _body0,
        out_shape=jax.ShapeDtypeStruct((PBUF, C3), ADT),
        scratch_shapes=[pltpu.VMEM((PBUF, C3), ADT)],
        interpret=interpret,
    )


def _call1(interpret=False):
    return pl.pallas_call(
        _body1,
        out_shape=jax.ShapeDtypeStruct((PBUF, C3), ADT),
        scratch_shapes=[pltpu.VMEM((PBUF, C3), ADT)],
        interpret=interpret,
    )


def _call2(interpret=False):
    return pl.pallas_call(
        _body2,
        out_shape=jax.ShapeDtypeStruct((B, H, W, C), jnp.float32),
        scratch_shapes=[pltpu.VMEM((PBUF, C3), ADT)],
        interpret=interpret,
    )


def _relayout(w):
    t = jnp.transpose(w.astype(ADT), (0, 3, 4, 2, 1))
    t = jnp.concatenate([t[:, :, 1:2], t[:, :, 0:1], t[:, :, 2:3]], axis=2)
    return t.reshape(2, 3, C3, C)


def _run(x, W_map, proj_W, proj_b, embed, res_w, res_b, gumbel,
         interpret=False):
    xt = jnp.transpose(x, (0, 2, 3, 1)).reshape(B, ROWS, C)
    wmap_t = W_map[:, :, 0, 0].T
    projw_t = proj_W[:, :, 0, 0].T
    pb = proj_b.reshape(1, NE)
    gum = gumbel[:, :, 0, 0]
    wks = [_relayout(res_w[i]) for i in range(3)]
    h = _call0(interpret)(xt, wmap_t, projw_t, pb, gum, embed,
                          wks[0], res_b[0])
    h = _call1(interpret)(h, wks[1], res_b[1])
    out = _call2(interpret)(h, wks[2], res_b[2])
    return jnp.transpose(out, (0, 3, 1, 2))


def kernel(x, W_map, proj_W, proj_b, embed, res_w, res_b, gumbel):
    return _run(x, W_map, proj_W, proj_b, embed, res_w, res_b, gumbel)


# speedup vs baseline: 1.1284x; 1.1284x over previous
import jax
import jax.numpy as jnp
from jax.experimental import pallas as pl
from jax.experimental.pallas import tpu as pltpu

C = 384
NE = 20
B = 8
H = 24
W = 24
ROWS = H * W
S = 640
G = 32
PBUF = B * S
ADT = jnp.bfloat16


def _edge_masks():
    sidx = jax.lax.broadcasted_iota(jnp.int32, (S, 1), 0)
    mask_m = jnp.where(sidx % W == (G % W), 0.0, 1.0).astype(ADT)
    mask_p = jnp.where(sidx % W == ((G - 1) % W), 0.0, 1.0).astype(ADT)
    return mask_m, mask_p


def _conv(src, dst, wk, kj, bias, mask_m, mask_p, resid_src=None,
          out_ref=None):
    for b in range(B):
        base = b * S
        slab = src[pl.ds(base, S), :]
        am = jnp.roll(slab, 1, axis=0) * mask_m
        ap = jnp.roll(slab, -1, axis=0) * mask_p
        taps = []
        for t in range(9):
            lo = G + W * (t // 3 - 1)
            sv = (am, slab, ap)[t % 3]
            taps.append(jax.lax.slice(sv, (lo, 0), (lo + ROWS, C)))
        lhs = jnp.concatenate(taps, axis=1)
        acc = jnp.dot(lhs, wk[kj], preferred_element_type=jnp.float32)
        val = acc + bias
        if resid_src is not None:
            val = val + resid_src[pl.ds(base + G, ROWS), :].astype(jnp.float32)
        val = jnp.maximum(val, 0.0)
        if out_ref is None:
            dst[pl.ds(base + G, ROWS), :] = val.astype(ADT)
        else:
            out_ref[b] = val.reshape(H, W, C)


def _body0(xt, wmap_t, projw_t, pb, gum, emb, wk, rb, h_out, r_scr):
    mask_m, mask_p = _edge_masks()
    h_out[...] = jnp.zeros((PBUF, C), ADT)
    r_scr[...] = jnp.zeros((PBUF, C), ADT)
    xv = xt[...].reshape(B, ROWS, C)
    pooled = jnp.mean(xv, axis=1)
    m = jax.nn.sigmoid(jnp.dot(pooled, wmap_t[...],
                               preferred_element_type=jnp.float32))
    logits = jnp.dot(m, projw_t[...],
                     preferred_element_type=jnp.float32) + pb[...]
    y = logits + gum[...]
    col = jax.lax.broadcasted_iota(jnp.int32, (B, NE), 1)
    ymax = jnp.max(y, axis=1, keepdims=True)
    amin = jnp.min(jnp.where(y == ymax, col, NE), axis=1, keepdims=True)
    oh = (col == amin).astype(jnp.float32)
    zq = jnp.dot(oh, emb[...], preferred_element_type=jnp.float32)
    v = xv + zq[:, None, :]
    for b in range(B):
        h_out[pl.ds(b * S + G, ROWS), :] = v[b].astype(ADT)
    _conv(h_out, r_scr, wk, 0, rb[0][None, :], mask_m, mask_p)
    _conv(r_scr, h_out, wk, 1, rb[1][None, :], mask_m, mask_p,
          resid_src=h_out)


def _body12(h_in, wk1, rb1, wk2, rb2, out, h_scr, r_scr):
    mask_m, mask_p = _edge_masks()
    h_scr[...] = jnp.zeros((PBUF, C), ADT)
    r_scr[...] = jnp.zeros((PBUF, C), ADT)
    _conv(h_in, r_scr, wk1, 0, rb1[0][None, :], mask_m, mask_p)
    _conv(r_scr, h_scr, wk1, 1, rb1[1][None, :], mask_m, mask_p,
          resid_src=h_in)
    _conv(h_scr, r_scr, wk2, 0, rb2[0][None, :], mask_m, mask_p)
    _conv(r_scr, None, wk2, 1, rb2[1][None, :], mask_m, mask_p,
          resid_src=h_scr, out_ref=out)


def _call0(interpret=False):
    return pl.pallas_call(
        _body0,
        out_shape=jax.ShapeDtypeStruct((PBUF, C), ADT),
        scratch_shapes=[pltpu.VMEM((PBUF, C), ADT)],
        interpret=interpret,
    )


def _call12(interpret=False):
    return pl.pallas_call(
        _body12,
        out_shape=jax.ShapeDtypeStruct((B, H, W, C), jnp.float32),
        scratch_shapes=[pltpu.VMEM((PBUF, C), ADT),
                        pltpu.VMEM((PBUF, C), ADT)],
        interpret=interpret,
    )


def _run(x, W_map, proj_W, proj_b, embed, res_w, res_b, gumbel,
         interpret=False):
    xt = jnp.transpose(x, (0, 2, 3, 1))
    wmap_t = W_map[:, :, 0, 0].T
    projw_t = proj_W[:, :, 0, 0].T
    pb = proj_b.reshape(1, NE)
    gum = gumbel[:, :, 0, 0]
    wks = [jnp.transpose(res_w[i].astype(ADT),
                         (0, 3, 4, 2, 1)).reshape(2, 9 * C, C)
           for i in range(3)]
    h = _call0(interpret)(xt, wmap_t, projw_t, pb, gum, embed,
                          wks[0], res_b[0])
    out = _call12(interpret)(h, wks[1], res_b[1], wks[2], res_b[2])
    return jnp.transpose(out, (0, 3, 1, 2))


def kernel(x, W_map, proj_W, proj_b, embed, res_w, res_b, gumbel):
    return _run(x, W_map, proj_W, proj_b, embed, res_w, res_b, gumbel)


# speedup vs baseline: 1.1374x; 1.0079x over previous
import jax
import jax.numpy as jnp
from jax.experimental import pallas as pl
from jax.experimental.pallas import tpu as pltpu

C = 384
NE = 20
B = 8
H = 24
W = 24
ROWS = H * W
S = 640
G = 32
PBUF = B * S
ADT = jnp.bfloat16


def _zero_pads(ref):
    z = jnp.zeros((G, C), ADT)
    for b in range(B):
        ref[pl.ds(b * S, G), :] = z
        ref[pl.ds(b * S + S - G, G), :] = z


def _edge_masks():
    sidx = jax.lax.broadcasted_iota(jnp.int32, (S, 1), 0)
    mask_m = jnp.where(sidx % W == (G % W), 0.0, 1.0).astype(ADT)
    mask_p = jnp.where(sidx % W == ((G - 1) % W), 0.0, 1.0).astype(ADT)
    return mask_m, mask_p


def _conv(src, dst, wk, kj, bias, mask_m, mask_p, resid_src=None,
          out_ref=None):
    for b in range(B):
        base = b * S
        slab = src[pl.ds(base, S), :]
        am = jnp.roll(slab, 1, axis=0) * mask_m
        ap = jnp.roll(slab, -1, axis=0) * mask_p
        taps = []
        for t in range(9):
            lo = G + W * (t // 3 - 1)
            sv = (am, slab, ap)[t % 3]
            taps.append(jax.lax.slice(sv, (lo, 0), (lo + ROWS, C)))
        lhs = jnp.concatenate(taps, axis=1)
        acc = jnp.dot(lhs, wk[kj], preferred_element_type=jnp.float32)
        val = acc + bias
        if resid_src is not None:
            val = val + resid_src[pl.ds(base + G, ROWS), :].astype(jnp.float32)
        val = jnp.maximum(val, 0.0)
        if out_ref is None:
            dst[pl.ds(base + G, ROWS), :] = val.astype(ADT)
        else:
            out_ref[b] = val.reshape(H, W, C)


def _body0(xt, wmap_t, projw_t, pb, gum, emb, wk, rb, h_out, r_scr):
    mask_m, mask_p = _edge_masks()
    _zero_pads(h_out)
    _zero_pads(r_scr)
    xv = xt[...].reshape(B, ROWS, C)
    pooled = jnp.mean(xv, axis=1)
    m = jax.nn.sigmoid(jnp.dot(pooled, wmap_t[...],
                               preferred_element_type=jnp.float32))
    logits = jnp.dot(m, projw_t[...],
                     preferred_element_type=jnp.float32) + pb[...]
    y = logits + gum[...]
    col = jax.lax.broadcasted_iota(jnp.int32, (B, NE), 1)
    ymax = jnp.max(y, axis=1, keepdims=True)
    amin = jnp.min(jnp.where(y == ymax, col, NE), axis=1, keepdims=True)
    oh = (col == amin).astype(jnp.float32)
    zq = jnp.dot(oh, emb[...], preferred_element_type=jnp.float32)
    v = xv + zq[:, None, :]
    for b in range(B):
        h_out[pl.ds(b * S + G, ROWS), :] = v[b].astype(ADT)
    _conv(h_out, r_scr, wk, 0, rb[0][None, :], mask_m, mask_p)
    _conv(r_scr, h_out, wk, 1, rb[1][None, :], mask_m, mask_p,
          resid_src=h_out)


def _body12(h_in, wk1, rb1, wk2, rb2, out, h_scr, r_scr):
    mask_m, mask_p = _edge_masks()
    _zero_pads(h_scr)
    _zero_pads(r_scr)
    _conv(h_in, r_scr, wk1, 0, rb1[0][None, :], mask_m, mask_p)
    _conv(r_scr, h_scr, wk1, 1, rb1[1][None, :], mask_m, mask_p,
          resid_src=h_in)
    _conv(h_scr, r_scr, wk2, 0, rb2[0][None, :], mask_m, mask_p)
    _conv(r_scr, None, wk2, 1, rb2[1][None, :], mask_m, mask_p,
          resid_src=h_scr, out_ref=out)


def _call0(interpret=False):
    return pl.pallas_call(
        _body0,
        out_shape=jax.ShapeDtypeStruct((PBUF, C), ADT),
        scratch_shapes=[pltpu.VMEM((PBUF, C), ADT)],
        interpret=interpret,
    )


def _call12(interpret=False):
    return pl.pallas_call(
        _body12,
        out_shape=jax.ShapeDtypeStruct((B, H, W, C), jnp.float32),
        scratch_shapes=[pltpu.VMEM((PBUF, C), ADT),
                        pltpu.VMEM((PBUF, C), ADT)],
        interpret=interpret,
    )


def _run(x, W_map, proj_W, proj_b, embed, res_w, res_b, gumbel,
         interpret=False):
    xt = jnp.transpose(x, (0, 2, 3, 1))
    wmap_t = W_map[:, :, 0, 0].T
    projw_t = proj_W[:, :, 0, 0].T
    pb = proj_b.reshape(1, NE)
    gum = gumbel[:, :, 0, 0]
    wks = [jnp.transpose(res_w[i].astype(ADT),
                         (0, 3, 4, 2, 1)).reshape(2, 9 * C, C)
           for i in range(3)]
    h = _call0(interpret)(xt, wmap_t, projw_t, pb, gum, embed,
                          wks[0], res_b[0])
    out = _call12(interpret)(h, wks[1], res_b[1], wks[2], res_b[2])
    return jnp.transpose(out, (0, 3, 1, 2))


def kernel(x, W_map, proj_W, proj_b, embed, res_w, res_b, gumbel):
    return _run(x, W_map, proj_W, proj_b, embed, res_w, res_b, gumbel)
